# Initial kernel scaffold; baseline (speedup 1.0000x reference)
import functools

import jax
import jax.numpy as jnp
from jax import lax
from jax.experimental import pallas as pl
from jax.experimental.pallas import tpu as pltpu
from jax.experimental.pallas import tpu_sc as plsc

_N_E = 8192
_E_DIM = 32
_N_TOKENS = 32768

_T = 256
_G = _N_TOKENS // _T

_NC = 2
_NS = 16
_NW = _NC * _NS
_B_PER_W = _N_TOKENS // _NW
_CHUNK = 128
_NCHUNK = _B_PER_W // _CHUNK


def _dist_argmin_body(x_ref, wt_ref, idx_ref):
    xb = x_ref[...]
    wt = wt_ref[...]
    xw = jnp.dot(xb, wt, preferred_element_type=jnp.float32)
    xsq = jnp.sum(xb * xb, axis=1, keepdims=True)
    wsq = jnp.sum(wt * wt, axis=0, keepdims=True)
    d = xsq + wsq - 2.0 * xw
    idx_ref[0, 0, :] = jnp.argmin(d, axis=1).astype(jnp.int32)


def _argmin_tc(x, wt):
    return pl.pallas_call(
        _dist_argmin_body,
        grid=(_G,),
        in_specs=[
            pl.BlockSpec((_T, _E_DIM), lambda i: (i, 0)),
            pl.BlockSpec((_E_DIM, _N_E), lambda i: (0, 0)),
        ],
        out_specs=pl.BlockSpec((1, 1, _T), lambda i: (i, 0, 0)),
        out_shape=jax.ShapeDtypeStruct((_G, 1, _T), jnp.int32),
    )(x, wt)


def _sc_gather_body(table_hbm, idx_hbm, out_hbm, idx_v, rows_v, sem):
    wid = lax.axis_index("s") * _NC + lax.axis_index("c")
    base = wid * _B_PER_W
    for j in range(_NCHUNK):
        pltpu.sync_copy(idx_hbm.at[pl.ds(base + j * _CHUNK, _CHUNK)],
                        idx_v.at[j])
    copies = []
    for j in range(_NCHUNK):
        copies.append(pltpu.async_copy(
            table_hbm.at[idx_v.at[j]],
            rows_v.at[pl.ds(j * _CHUNK, _CHUNK)], sem))
    for c in copies:
        c.wait()
    pltpu.sync_copy(rows_v, out_hbm.at[pl.ds(base, _B_PER_W)])


_sc_gather = functools.partial(
    pl.kernel,
    _sc_gather_body,
    out_type=jax.ShapeDtypeStruct((_N_TOKENS, _E_DIM), jnp.float32),
    mesh=plsc.VectorSubcoreMesh(core_axis_name="c", subcore_axis_name="s"),
    scratch_types=[
        pltpu.VMEM((_NCHUNK, _CHUNK), jnp.int32),
        pltpu.VMEM((_B_PER_W, _E_DIM), jnp.float32),
        pltpu.SemaphoreType.DMA,
    ],
)()


def kernel(x, W):
    wt = W.T
    idx3 = _argmin_tc(x, wt)
    min_indexes = idx3.reshape(_N_TOKENS)
    z_q = _sc_gather(W, min_indexes)
    return (z_q, min_indexes)


# speedup vs baseline: 1.3990x; 1.3990x over previous
import functools

import jax
import jax.numpy as jnp
from jax import lax
from jax.experimental import pallas as pl
from jax.experimental.pallas import tpu as pltpu
from jax.experimental.pallas import tpu_sc as plsc

_N_E = 8192
_E_DIM = 32
_N_TOKENS = 32768

_T = 256
_G = _N_TOKENS // _T

_NC = 2
_NS = 16
_NW = _NC * _NS
_B_PER_W = _N_TOKENS // _NW
_CHUNK = 128
_NCHUNK = _B_PER_W // _CHUNK


def _dist_argmin_body(x_ref, wt_ref, idx_ref):
    xb = x_ref[...]
    wt = wt_ref[...]
    xw = jnp.dot(xb, wt, preferred_element_type=jnp.float32)
    xsq = jnp.sum(xb * xb, axis=1, keepdims=True)
    wsq = jnp.sum(wt * wt, axis=0, keepdims=True)
    d = xsq + wsq - 2.0 * xw
    dmin = jnp.min(d, axis=1, keepdims=True)
    ii = lax.broadcasted_iota(jnp.int32, (_T, _N_E), 1)
    idx = jnp.min(jnp.where(d == dmin, ii, _N_E), axis=1)
    idx_ref[0, 0, :] = idx.astype(jnp.int32)


def _argmin_tc(x, wt):
    return pl.pallas_call(
        _dist_argmin_body,
        grid=(_G,),
        in_specs=[
            pl.BlockSpec((_T, _E_DIM), lambda i: (i, 0)),
            pl.BlockSpec((_E_DIM, _N_E), lambda i: (0, 0)),
        ],
        out_specs=pl.BlockSpec((1, 1, _T), lambda i: (i, 0, 0)),
        out_shape=jax.ShapeDtypeStruct((_G, 1, _T), jnp.int32),
    )(x, wt)


def _sc_gather_body(table_hbm, idx_hbm, out_hbm, idx_v, rows_v, sem):
    wid = lax.axis_index("s") * _NC + lax.axis_index("c")
    base = wid * _B_PER_W
    for j in range(_NCHUNK):
        pltpu.sync_copy(idx_hbm.at[pl.ds(base + j * _CHUNK, _CHUNK)],
                        idx_v.at[j])
    copies = []
    for j in range(_NCHUNK):
        copies.append(pltpu.async_copy(
            table_hbm.at[idx_v.at[j]],
            rows_v.at[pl.ds(j * _CHUNK, _CHUNK)], sem))
    for c in copies:
        c.wait()
    pltpu.sync_copy(rows_v, out_hbm.at[pl.ds(base, _B_PER_W)])


@functools.cache
def _sc_gather():
    return pl.kernel(
        _sc_gather_body,
        out_type=jax.ShapeDtypeStruct((_N_TOKENS, _E_DIM), jnp.float32),
        mesh=plsc.VectorSubcoreMesh(core_axis_name="c", subcore_axis_name="s"),
        scratch_types=[
            pltpu.VMEM((_NCHUNK, _CHUNK), jnp.int32),
            pltpu.VMEM((_B_PER_W, _E_DIM), jnp.float32),
            pltpu.SemaphoreType.DMA,
        ],
        compiler_params=pltpu.CompilerParams(use_tc_tiling_on_sc=False),
    )


def kernel(x, W):
    wt = W.T
    idx3 = _argmin_tc(x, wt)
    min_indexes = idx3.reshape(_N_TOKENS)
    z_q = _sc_gather()(W, min_indexes)
    return (z_q, min_indexes)


# speedup vs baseline: 1.7331x; 1.2389x over previous
import functools

import jax
import jax.numpy as jnp
from jax import lax
from jax.experimental import pallas as pl
from jax.experimental.pallas import tpu as pltpu
from jax.experimental.pallas import tpu_sc as plsc

_N_E = 8192
_E_DIM = 32
_N_TOKENS = 32768

_T = 256
_G = _N_TOKENS // _T

_NC = 2
_NS = 16
_NW = _NC * _NS
_B_PER_W = _N_TOKENS // _NW
_CHUNK = 128
_NCHUNK = _B_PER_W // _CHUNK


_C = 128


def _dist_argmin_body(x_ref, wtn_ref, idx_ref):
    xb = x_ref[...]
    wtn = wtn_ref[...]
    xwn = jnp.dot(xb, wtn, preferred_element_type=jnp.float32)
    xsq = jnp.sum(xb * xb, axis=1, keepdims=True)
    wsq = 0.25 * jnp.sum(wtn * wtn, axis=0, keepdims=True)
    run_min = jnp.zeros((_T, _C), jnp.float32)
    run_c = jnp.zeros((_T, _C), jnp.float32)
    for c in range(_N_E // _C):
        d_c = (xsq + wsq[:, c * _C:(c + 1) * _C]) + xwn[:, c * _C:(c + 1) * _C]
        if c == 0:
            run_min = d_c
        else:
            mask = d_c < run_min
            run_min = jnp.where(mask, d_c, run_min)
            run_c = jnp.where(mask, jnp.float32(c), run_c)
    lane = lax.broadcasted_iota(jnp.int32, (_T, _C), 1).astype(jnp.float32)
    jf = run_c * jnp.float32(_C) + lane
    dmin = jnp.min(run_min, axis=1, keepdims=True)
    cand = jnp.where(run_min == dmin, jf, jnp.float32(_N_E))
    idx_ref[0, 0, :] = jnp.min(cand, axis=1).astype(jnp.int32)


def _argmin_tc(x, wt):
    return pl.pallas_call(
        _dist_argmin_body,
        grid=(_G,),
        in_specs=[
            pl.BlockSpec((_T, _E_DIM), lambda i: (i, 0)),
            pl.BlockSpec((_E_DIM, _N_E), lambda i: (0, 0)),
        ],
        out_specs=pl.BlockSpec((1, 1, _T), lambda i: (i, 0, 0)),
        out_shape=jax.ShapeDtypeStruct((_G, 1, _T), jnp.int32),
    )(x, wt)


def _sc_gather_body(table_hbm, idx_hbm, out_hbm, idx_v, rows_v, sem):
    wid = lax.axis_index("s") * _NC + lax.axis_index("c")
    base = wid * _B_PER_W
    for j in range(_NCHUNK):
        pltpu.sync_copy(idx_hbm.at[pl.ds(base + j * _CHUNK, _CHUNK)],
                        idx_v.at[j])
    copies = []
    for j in range(_NCHUNK):
        copies.append(pltpu.async_copy(
            table_hbm.at[idx_v.at[j]],
            rows_v.at[pl.ds(j * _CHUNK, _CHUNK)], sem))
    for c in copies:
        c.wait()
    pltpu.sync_copy(rows_v, out_hbm.at[pl.ds(base, _B_PER_W)])


@functools.cache
def _sc_gather():
    return pl.kernel(
        _sc_gather_body,
        out_type=jax.ShapeDtypeStruct((_N_TOKENS, _E_DIM), jnp.float32),
        mesh=plsc.VectorSubcoreMesh(core_axis_name="c", subcore_axis_name="s"),
        scratch_types=[
            pltpu.VMEM((_NCHUNK, _CHUNK), jnp.int32),
            pltpu.VMEM((_B_PER_W, _E_DIM), jnp.float32),
            pltpu.SemaphoreType.DMA,
        ],
        compiler_params=pltpu.CompilerParams(use_tc_tiling_on_sc=False),
    )


def kernel(x, W):
    wt = W.T * jnp.float32(-2.0)
    idx3 = _argmin_tc(x, wt)
    min_indexes = idx3.reshape(_N_TOKENS)
    z_q = _sc_gather()(W, min_indexes)
    return (z_q, min_indexes)


# speedup vs baseline: 1.8247x; 1.0529x over previous
import functools

import jax
import jax.numpy as jnp
from jax import lax
from jax.experimental import pallas as pl
from jax.experimental.pallas import tpu as pltpu
from jax.experimental.pallas import tpu_sc as plsc

_N_E = 8192
_E_DIM = 32
_N_TOKENS = 32768

_T = 512
_G = _N_TOKENS // _T

_NC = 2
_NS = 16
_NW = _NC * _NS
_B_PER_W = _N_TOKENS // _NW
_CHUNK = 128
_NCHUNK = _B_PER_W // _CHUNK


_C = 128


def _dist_argmin_body(x_ref, wtn_ref, idx_ref):
    xb = x_ref[...]
    wtn = wtn_ref[...]
    xwn = jnp.dot(xb, wtn, preferred_element_type=jnp.float32)
    xsq = jnp.sum(xb * xb, axis=1, keepdims=True)
    wsq = 0.25 * jnp.sum(wtn * wtn, axis=0, keepdims=True)
    run_min = jnp.zeros((_T, _C), jnp.float32)
    run_c = jnp.zeros((_T, _C), jnp.float32)
    for c in range(_N_E // _C):
        d_c = (xsq + wsq[:, c * _C:(c + 1) * _C]) + xwn[:, c * _C:(c + 1) * _C]
        if c == 0:
            run_min = d_c
        else:
            mask = d_c < run_min
            run_min = jnp.where(mask, d_c, run_min)
            run_c = jnp.where(mask, jnp.float32(c), run_c)
    lane = lax.broadcasted_iota(jnp.int32, (_T, _C), 1).astype(jnp.float32)
    jf = run_c * jnp.float32(_C) + lane
    dmin = jnp.min(run_min, axis=1, keepdims=True)
    cand = jnp.where(run_min == dmin, jf, jnp.float32(_N_E))
    idx_ref[0, 0, :] = jnp.min(cand, axis=1).astype(jnp.int32)


def _argmin_tc(x, wt):
    return pl.pallas_call(
        _dist_argmin_body,
        grid=(_G,),
        in_specs=[
            pl.BlockSpec((_T, _E_DIM), lambda i: (i, 0)),
            pl.BlockSpec((_E_DIM, _N_E), lambda i: (0, 0)),
        ],
        out_specs=pl.BlockSpec((1, 1, _T), lambda i: (i, 0, 0)),
        out_shape=jax.ShapeDtypeStruct((_G, 1, _T), jnp.int32),
    )(x, wt)


def _sc_gather_body(table_hbm, idx_hbm, out_hbm, idx_v, rows_v, sem):
    wid = lax.axis_index("s") * _NC + lax.axis_index("c")
    base = wid * _B_PER_W
    for j in range(_NCHUNK):
        pltpu.sync_copy(idx_hbm.at[pl.ds(base + j * _CHUNK, _CHUNK)],
                        idx_v.at[j])
    copies = []
    for j in range(_NCHUNK):
        copies.append(pltpu.async_copy(
            table_hbm.at[idx_v.at[j]],
            rows_v.at[pl.ds(j * _CHUNK, _CHUNK)], sem))
    for c in copies:
        c.wait()
    pltpu.sync_copy(rows_v, out_hbm.at[pl.ds(base, _B_PER_W)])


@functools.cache
def _sc_gather():
    return pl.kernel(
        _sc_gather_body,
        out_type=jax.ShapeDtypeStruct((_N_TOKENS, _E_DIM), jnp.float32),
        mesh=plsc.VectorSubcoreMesh(core_axis_name="c", subcore_axis_name="s"),
        scratch_types=[
            pltpu.VMEM((_NCHUNK, _CHUNK), jnp.int32),
            pltpu.VMEM((_B_PER_W, _E_DIM), jnp.float32),
            pltpu.SemaphoreType.DMA,
        ],
        compiler_params=pltpu.CompilerParams(use_tc_tiling_on_sc=False),
    )


def kernel(x, W):
    wt = W.T * jnp.float32(-2.0)
    idx3 = _argmin_tc(x, wt)
    min_indexes = idx3.reshape(_N_TOKENS)
    z_q = _sc_gather()(W, min_indexes)
    return (z_q, min_indexes)


# speedup vs baseline: 1.8285x; 1.0021x over previous
import functools

import jax
import jax.numpy as jnp
from jax import lax
from jax.experimental import pallas as pl
from jax.experimental.pallas import tpu as pltpu
from jax.experimental.pallas import tpu_sc as plsc

_N_E = 8192
_E_DIM = 32
_N_TOKENS = 32768

_T = 512
_G = _N_TOKENS // _T

_NC = 2
_NS = 16
_NW = _NC * _NS
_B_PER_W = _N_TOKENS // _NW
_CHUNK = 128
_NCHUNK = _B_PER_W // _CHUNK


_C = 128


def _dist_argmin_body(x_ref, wtn_ref, idx_ref):
    xb = x_ref[...]
    wtn = wtn_ref[...]
    xwn = jnp.dot(xb, wtn, preferred_element_type=jnp.float32)
    xsq = jnp.sum(xb * xb, axis=1, keepdims=True)
    wsq = 0.25 * jnp.sum(wtn * wtn, axis=0, keepdims=True)
    run_min = jnp.zeros((_T, _C), jnp.float32)
    run_c = jnp.zeros((_T, _C), jnp.float32)
    for c in range(_N_E // _C):
        d_c = (xsq + wsq[:, c * _C:(c + 1) * _C]) + xwn[:, c * _C:(c + 1) * _C]
        if c == 0:
            run_min = d_c
        else:
            mask = d_c < run_min
            run_min = jnp.where(mask, d_c, run_min)
            run_c = jnp.where(mask, jnp.float32(c), run_c)
    lane = lax.broadcasted_iota(jnp.int32, (_T, _C), 1).astype(jnp.float32)
    jf = run_c * jnp.float32(_C) + lane
    dmin = jnp.min(run_min, axis=1, keepdims=True)
    cand = jnp.where(run_min == dmin, jf, jnp.float32(_N_E))
    idx_ref[...] = jnp.min(cand, axis=1).astype(jnp.int32)


def _argmin_tc(x, wt):
    return pl.pallas_call(
        _dist_argmin_body,
        grid=(_G,),
        in_specs=[
            pl.BlockSpec((_T, _E_DIM), lambda i: (i, 0)),
            pl.BlockSpec((_E_DIM, _N_E), lambda i: (0, 0)),
        ],
        out_specs=pl.BlockSpec((_T,), lambda i: (i,)),
        out_shape=jax.ShapeDtypeStruct((_N_TOKENS,), jnp.int32),
    )(x, wt)


def _sc_gather_body(table_hbm, idx_hbm, out_hbm, idx_v, rows_v, sem):
    wid = lax.axis_index("s") * _NC + lax.axis_index("c")
    base = wid * _B_PER_W
    for j in range(_NCHUNK):
        pltpu.sync_copy(idx_hbm.at[pl.ds(base + j * _CHUNK, _CHUNK)],
                        idx_v.at[j])
    copies = []
    for j in range(_NCHUNK):
        copies.append(pltpu.async_copy(
            table_hbm.at[idx_v.at[j]],
            rows_v.at[pl.ds(j * _CHUNK, _CHUNK)], sem))
    for c in copies:
        c.wait()
    pltpu.sync_copy(rows_v, out_hbm.at[pl.ds(base, _B_PER_W)])


@functools.cache
def _sc_gather():
    return pl.kernel(
        _sc_gather_body,
        out_type=jax.ShapeDtypeStruct((_N_TOKENS, _E_DIM), jnp.float32),
        mesh=plsc.VectorSubcoreMesh(core_axis_name="c", subcore_axis_name="s"),
        scratch_types=[
            pltpu.VMEM((_NCHUNK, _CHUNK), jnp.int32),
            pltpu.VMEM((_B_PER_W, _E_DIM), jnp.float32),
            pltpu.SemaphoreType.DMA,
        ],
        compiler_params=pltpu.CompilerParams(use_tc_tiling_on_sc=False),
    )


def kernel(x, W):
    wt = W.T * jnp.float32(-2.0)
    min_indexes = _argmin_tc(x, wt)
    z_q = _sc_gather()(W, min_indexes)
    return (z_q, min_indexes)


# speedup vs baseline: 1.8289x; 1.0002x over previous
import functools

import jax
import jax.numpy as jnp
from jax import lax
from jax.experimental import pallas as pl
from jax.experimental.pallas import tpu as pltpu
from jax.experimental.pallas import tpu_sc as plsc

_N_E = 8192
_E_DIM = 32
_N_TOKENS = 32768

_T = 512
_G = _N_TOKENS // _T

_NC = 2
_NS = 16
_NW = _NC * _NS
_B_PER_W = _N_TOKENS // _NW
_CHUNK = 128
_NCHUNK = _B_PER_W // _CHUNK


_C = 128


def _dist_argmin_body(x_ref, w_ref, idx_ref, wtn_ref):
    @pl.when(pl.program_id(0) == 0)
    def _init():
        wtn_ref[...] = jnp.transpose(w_ref[...]) * jnp.float32(-2.0)

    xb = x_ref[...]
    wtn = wtn_ref[...]
    xwn = jnp.dot(xb, wtn, preferred_element_type=jnp.float32)
    xsq = jnp.sum(xb * xb, axis=1, keepdims=True)
    wsq = 0.25 * jnp.sum(wtn * wtn, axis=0, keepdims=True)
    run_min = jnp.zeros((_T, _C), jnp.float32)
    run_c = jnp.zeros((_T, _C), jnp.float32)
    for c in range(_N_E // _C):
        d_c = (xsq + wsq[:, c * _C:(c + 1) * _C]) + xwn[:, c * _C:(c + 1) * _C]
        if c == 0:
            run_min = d_c
        else:
            mask = d_c < run_min
            run_min = jnp.where(mask, d_c, run_min)
            run_c = jnp.where(mask, jnp.float32(c), run_c)
    lane = lax.broadcasted_iota(jnp.int32, (_T, _C), 1).astype(jnp.float32)
    jf = run_c * jnp.float32(_C) + lane
    dmin = jnp.min(run_min, axis=1, keepdims=True)
    cand = jnp.where(run_min == dmin, jf, jnp.float32(_N_E))
    idx_ref[...] = jnp.min(cand, axis=1).astype(jnp.int32)


def _argmin_tc(x, wt):
    return pl.pallas_call(
        _dist_argmin_body,
        grid=(_G,),
        in_specs=[
            pl.BlockSpec((_T, _E_DIM), lambda i: (i, 0)),
            pl.BlockSpec((_N_E, _E_DIM), lambda i: (0, 0)),
        ],
        out_specs=pl.BlockSpec((_T,), lambda i: (i,)),
        out_shape=jax.ShapeDtypeStruct((_N_TOKENS,), jnp.int32),
        scratch_shapes=[pltpu.VMEM((_E_DIM, _N_E), jnp.float32)],
    )(x, wt)


def _sc_gather_body(table_hbm, idx_hbm, out_hbm, idx_v, rows_v, sem):
    wid = lax.axis_index("s") * _NC + lax.axis_index("c")
    base = wid * _B_PER_W
    for j in range(_NCHUNK):
        pltpu.sync_copy(idx_hbm.at[pl.ds(base + j * _CHUNK, _CHUNK)],
                        idx_v.at[j])
    copies = []
    for j in range(_NCHUNK):
        copies.append(pltpu.async_copy(
            table_hbm.at[idx_v.at[j]],
            rows_v.at[pl.ds(j * _CHUNK, _CHUNK)], sem))
    for c in copies:
        c.wait()
    pltpu.sync_copy(rows_v, out_hbm.at[pl.ds(base, _B_PER_W)])


@functools.cache
def _sc_gather():
    return pl.kernel(
        _sc_gather_body,
        out_type=jax.ShapeDtypeStruct((_N_TOKENS, _E_DIM), jnp.float32),
        mesh=plsc.VectorSubcoreMesh(core_axis_name="c", subcore_axis_name="s"),
        scratch_types=[
            pltpu.VMEM((_NCHUNK, _CHUNK), jnp.int32),
            pltpu.VMEM((_B_PER_W, _E_DIM), jnp.float32),
            pltpu.SemaphoreType.DMA,
        ],
        compiler_params=pltpu.CompilerParams(use_tc_tiling_on_sc=False),
    )


def kernel(x, W):
    min_indexes = _argmin_tc(x, W)
    z_q = _sc_gather()(W, min_indexes)
    return (z_q, min_indexes)


# speedup vs baseline: 1.8580x; 1.0159x over previous
import functools

import jax
import jax.numpy as jnp
from jax import lax
from jax.experimental import pallas as pl
from jax.experimental.pallas import tpu as pltpu
from jax.experimental.pallas import tpu_sc as plsc

_N_E = 8192
_E_DIM = 32
_N_TOKENS = 32768

_T = 512
_G = _N_TOKENS // _T

_NC = 2
_NS = 16
_NW = _NC * _NS
_B_PER_W = _N_TOKENS // _NW
_CHUNK = 128
_NCHUNK = _B_PER_W // _CHUNK


_C = 128


def _dist_argmin_body(x_ref, w_ref, idx_ref, wtn_ref, wsq_ref):
    @pl.when(pl.program_id(0) == 0)
    def _init():
        wtn0 = jnp.transpose(w_ref[...]) * jnp.float32(-2.0)
        wtn_ref[...] = wtn0
        wsq_ref[...] = 0.25 * jnp.sum(wtn0 * wtn0, axis=0, keepdims=True)

    xb = x_ref[...]
    wtn = wtn_ref[...]
    xwn = jnp.dot(xb, wtn, preferred_element_type=jnp.float32)
    xsq = jnp.sum(xb * xb, axis=1, keepdims=True)
    wsq = wsq_ref[...]
    run_min = jnp.zeros((_T, _C), jnp.float32)
    run_c = jnp.zeros((_T, _C), jnp.float32)
    for c in range(_N_E // _C):
        d_c = (xsq + wsq[:, c * _C:(c + 1) * _C]) + xwn[:, c * _C:(c + 1) * _C]
        if c == 0:
            run_min = d_c
        else:
            mask = d_c < run_min
            run_min = jnp.where(mask, d_c, run_min)
            run_c = jnp.where(mask, jnp.float32(c), run_c)
    lane = lax.broadcasted_iota(jnp.int32, (_T, _C), 1).astype(jnp.float32)
    jf = run_c * jnp.float32(_C) + lane
    dmin = jnp.min(run_min, axis=1, keepdims=True)
    cand = jnp.where(run_min == dmin, jf, jnp.float32(_N_E))
    idx_ref[...] = jnp.min(cand, axis=1).astype(jnp.int32)


def _argmin_tc(x, wt):
    return pl.pallas_call(
        _dist_argmin_body,
        grid=(_G,),
        in_specs=[
            pl.BlockSpec((_T, _E_DIM), lambda i: (i, 0)),
            pl.BlockSpec((_N_E, _E_DIM), lambda i: (0, 0)),
        ],
        out_specs=pl.BlockSpec((_T,), lambda i: (i,)),
        out_shape=jax.ShapeDtypeStruct((_N_TOKENS,), jnp.int32),
        scratch_shapes=[pltpu.VMEM((_E_DIM, _N_E), jnp.float32),
                        pltpu.VMEM((1, _N_E), jnp.float32)],
    )(x, wt)


def _sc_gather_body(table_hbm, idx_hbm, out_hbm, idx_v, rows_v, sem):
    wid = lax.axis_index("s") * _NC + lax.axis_index("c")
    base = wid * _B_PER_W
    for j in range(_NCHUNK):
        pltpu.sync_copy(idx_hbm.at[pl.ds(base + j * _CHUNK, _CHUNK)],
                        idx_v.at[j])
    copies = []
    for j in range(_NCHUNK):
        copies.append(pltpu.async_copy(
            table_hbm.at[idx_v.at[j]],
            rows_v.at[pl.ds(j * _CHUNK, _CHUNK)], sem))
    for c in copies:
        c.wait()
    pltpu.sync_copy(rows_v, out_hbm.at[pl.ds(base, _B_PER_W)])


@functools.cache
def _sc_gather():
    return pl.kernel(
        _sc_gather_body,
        out_type=jax.ShapeDtypeStruct((_N_TOKENS, _E_DIM), jnp.float32),
        mesh=plsc.VectorSubcoreMesh(core_axis_name="c", subcore_axis_name="s"),
        scratch_types=[
            pltpu.VMEM((_NCHUNK, _CHUNK), jnp.int32),
            pltpu.VMEM((_B_PER_W, _E_DIM), jnp.float32),
            pltpu.SemaphoreType.DMA,
        ],
        compiler_params=pltpu.CompilerParams(use_tc_tiling_on_sc=False),
    )


def kernel(x, W):
    min_indexes = _argmin_tc(x, W)
    z_q = _sc_gather()(W, min_indexes)
    return (z_q, min_indexes)
